# Initial kernel scaffold; baseline (speedup 1.0000x reference)
import jax
import jax.numpy as jnp
from jax.experimental import pallas as pl


def kernel(ids, table):
    raise NotImplementedError("write your pallas kernel here")


# speedup vs baseline: 4.8104x; 4.8104x over previous
import functools

import jax
import jax.numpy as jnp
from jax import lax
from jax.experimental import pallas as pl
from jax.experimental.pallas import tpu as pltpu
from jax.experimental.pallas import tpu_sc as plsc

NUM_CARDS = 1000000
EMBED_DIM = 32
BATCH = 16384
HIST = 200

GROUP = 128
NUM_GROUPS = BATCH * HIST // GROUP
NW = 32
GROUPS_PER_W = NUM_GROUPS // NW
K = 8
STEPS = GROUPS_PER_W // K


def _embed_kernel(ids_hbm, table_hbm, out_hbm, idx_v, rows_v, sem):
    wid = lax.axis_index("s") * 2 + lax.axis_index("c")
    base0 = wid * GROUPS_PER_W

    def step(g, carry):
        base = base0 + g * K
        pltpu.sync_copy(ids_hbm.at[pl.ds(base, K)], idx_v)
        copies = [
            pltpu.async_copy(table_hbm.at[idx_v.at[j]], rows_v.at[j], sem)
            for j in range(K)
        ]
        for c in copies:
            c.wait()
        pltpu.sync_copy(rows_v, out_hbm.at[pl.ds(base, K)])
        return carry

    lax.fori_loop(0, STEPS, step, 0)


@jax.jit
def _embed(ids2d, table):
    fn = functools.partial(
        pl.kernel,
        out_type=jax.ShapeDtypeStruct((NUM_GROUPS, GROUP, EMBED_DIM), jnp.float32),
        mesh=plsc.VectorSubcoreMesh(core_axis_name="c", subcore_axis_name="s"),
        scratch_types=[
            pltpu.VMEM((K, GROUP), jnp.int32),
            pltpu.VMEM((K, GROUP, EMBED_DIM), jnp.float32),
            pltpu.SemaphoreType.DMA,
        ],
        compiler_params=pltpu.CompilerParams(use_tc_tiling_on_sc=False),
    )(_embed_kernel)
    return fn(ids2d, table)


def kernel(ids, table):
    ids2d = ids.astype(jnp.int32).reshape(NUM_GROUPS, GROUP)
    out = _embed(ids2d, table)
    return out.reshape(BATCH, HIST, EMBED_DIM)


# speedup vs baseline: 5.0256x; 1.0447x over previous
import functools

import jax
import jax.numpy as jnp
from jax import lax
from jax.experimental import pallas as pl
from jax.experimental.pallas import tpu as pltpu
from jax.experimental.pallas import tpu_sc as plsc

NUM_CARDS = 1000000
EMBED_DIM = 32
BATCH = 16384
HIST = 200

GROUP = 128
NUM_GROUPS = BATCH * HIST // GROUP
NW = 32
GROUPS_PER_W = NUM_GROUPS // NW
K = 4
NBUF = 4
STEPS = GROUPS_PER_W // K
OUTER = STEPS // NBUF


def _embed_kernel(ids_hbm, table_hbm, out_hbm, idx_v, rows_v, sem_i, sem_g, sem_o):
    wid = lax.axis_index("s") * 2 + lax.axis_index("c")
    base0 = wid * GROUPS_PER_W

    def idx_copy(g, b):
        return pltpu.make_async_copy(
            ids_hbm.at[pl.ds(base0 + g * K, K)], idx_v.at[b], sem_i.at[b])

    def out_copy(g, b):
        return pltpu.make_async_copy(
            rows_v.at[b], out_hbm.at[pl.ds(base0 + g * K, K)], sem_o.at[b])

    def gather(b, j):
        return pltpu.make_async_copy(
            table_hbm.at[idx_v.at[b].at[j]], rows_v.at[b].at[j], sem_g.at[b])

    def drain_and_flush(g, b):
        for j in range(K):
            gather(b, j).wait()
        out_copy(g, b).start()

    idx_copy(0, 0).start()

    def outer(p, carry):
        for u in range(NBUF):
            g = p * NBUF + u
            prev = (u - 1) % NBUF

            @pl.when(p >= 1)
            def _():
                out_copy(g - NBUF, u).wait()

            idx_copy(g, u).wait()
            for j in range(K):
                gather(u, j).start()

            if u > 0:
                drain_and_flush(g - 1, prev)
            else:
                @pl.when(p >= 1)
                def _():
                    drain_and_flush(g - 1, prev)

            if u < NBUF - 1:
                idx_copy(g + 1, u + 1).start()
            else:
                @pl.when(p < OUTER - 1)
                def _():
                    idx_copy(g + 1, 0).start()
        return carry

    lax.fori_loop(0, OUTER, outer, 0)

    drain_and_flush(STEPS - 1, NBUF - 1)
    for b in range(NBUF):
        out_copy(STEPS - NBUF + b, b).wait()


@jax.jit
def _embed(ids2d, table):
    fn = functools.partial(
        pl.kernel,
        out_type=jax.ShapeDtypeStruct((NUM_GROUPS, GROUP, EMBED_DIM), jnp.float32),
        mesh=plsc.VectorSubcoreMesh(core_axis_name="c", subcore_axis_name="s"),
        scratch_types=[
            pltpu.VMEM((NBUF, K, GROUP), jnp.int32),
            pltpu.VMEM((NBUF, K, GROUP, EMBED_DIM), jnp.float32),
            pltpu.SemaphoreType.DMA((NBUF,)),
            pltpu.SemaphoreType.DMA((NBUF,)),
            pltpu.SemaphoreType.DMA((NBUF,)),
        ],
        compiler_params=pltpu.CompilerParams(use_tc_tiling_on_sc=False),
    )(_embed_kernel)
    return fn(ids2d, table)


def kernel(ids, table):
    ids2d = ids.astype(jnp.int32).reshape(NUM_GROUPS, GROUP)
    out = _embed(ids2d, table)
    return out.reshape(BATCH, HIST, EMBED_DIM)


# speedup vs baseline: 5.0484x; 1.0045x over previous
import functools

import jax
import jax.numpy as jnp
from jax import lax
from jax.experimental import pallas as pl
from jax.experimental.pallas import tpu as pltpu
from jax.experimental.pallas import tpu_sc as plsc

NUM_CARDS = 1000000
EMBED_DIM = 32
BATCH = 16384
HIST = 200

NW = 32
ROWS_PER_W = BATCH // NW
K = 4
NBUF = 4
STEPS = ROWS_PER_W // K
OUTER = STEPS // NBUF
SPLITS = ((0, 128), (128, 72))


def _embed_kernel(ids_hbm, table_hbm, out_hbm, idx_v, rows_v, sem_i, sem_g, sem_o):
    wid = lax.axis_index("s") * 2 + lax.axis_index("c")
    base0 = wid * ROWS_PER_W

    def idx_copy(g, b):
        return pltpu.make_async_copy(
            ids_hbm.at[pl.ds(base0 + g * K, K)], idx_v.at[b], sem_i.at[b])

    def out_copy(g, b):
        return pltpu.make_async_copy(
            rows_v.at[b], out_hbm.at[pl.ds(base0 + g * K, K)], sem_o.at[b])

    def gathers(b):
        for j in range(K):
            for off, n in SPLITS:
                yield pltpu.make_async_copy(
                    table_hbm.at[idx_v.at[b].at[j].at[pl.ds(off, n)]],
                    rows_v.at[b].at[j].at[pl.ds(off, n)],
                    sem_g.at[b])

    def drain_and_flush(g, b):
        for c in gathers(b):
            c.wait()
        out_copy(g, b).start()

    idx_copy(0, 0).start()

    def outer(p, carry):
        for u in range(NBUF):
            g = p * NBUF + u
            prev = (u - 1) % NBUF

            @pl.when(p >= 1)
            def _():
                out_copy(g - NBUF, u).wait()

            idx_copy(g, u).wait()
            for c in gathers(u):
                c.start()

            if u > 0:
                drain_and_flush(g - 1, prev)
            else:
                @pl.when(p >= 1)
                def _():
                    drain_and_flush(g - 1, prev)

            if u < NBUF - 1:
                idx_copy(g + 1, u + 1).start()
            else:
                @pl.when(p < OUTER - 1)
                def _():
                    idx_copy(g + 1, 0).start()
        return carry

    lax.fori_loop(0, OUTER, outer, 0)

    drain_and_flush(STEPS - 1, NBUF - 1)
    for b in range(NBUF):
        out_copy(STEPS - NBUF + b, b).wait()


@jax.jit
def _embed(ids, table):
    fn = functools.partial(
        pl.kernel,
        out_type=jax.ShapeDtypeStruct((BATCH, HIST, EMBED_DIM), jnp.float32),
        mesh=plsc.VectorSubcoreMesh(core_axis_name="c", subcore_axis_name="s"),
        scratch_types=[
            pltpu.VMEM((NBUF, K, HIST), jnp.int32),
            pltpu.VMEM((NBUF, K, HIST, EMBED_DIM), jnp.float32),
            pltpu.SemaphoreType.DMA((NBUF,)),
            pltpu.SemaphoreType.DMA((NBUF,)),
            pltpu.SemaphoreType.DMA((NBUF,)),
        ],
        compiler_params=pltpu.CompilerParams(use_tc_tiling_on_sc=False),
    )(_embed_kernel)
    return fn(ids, table)


def kernel(ids, table):
    return _embed(ids.astype(jnp.int32), table)
